# baseline (speedup 1.0000x reference)
import functools

import jax
import jax.numpy as jnp
from jax import lax
from jax.experimental import pallas as pl
from jax.experimental.pallas import tpu as pltpu
from jax.experimental.pallas import tpu_sc as plsc

B = 16384
EMB = 64
ITEM_EMB = 24
NC, NS = 2, 16
NW = NC * NS
BPW = B // NW
CH = 128
NCH = BPW // CH

_sc_mesh = plsc.VectorSubcoreMesh(core_axis_name="c", subcore_axis_name="s")


@functools.partial(
    pl.kernel,
    out_type=(
        jax.ShapeDtypeStruct((B, EMB), jnp.float32),
        jax.ShapeDtypeStruct((B, ITEM_EMB), jnp.float32),
    ),
    mesh=_sc_mesh,
    compiler_params=pltpu.CompilerParams(use_tc_tiling_on_sc=False),
    scratch_types=[
        pltpu.VMEM((NCH, CH), jnp.int32),
        pltpu.VMEM((NCH, CH), jnp.int32),
        pltpu.VMEM((BPW, EMB), jnp.float32),
        pltpu.VMEM((BPW, ITEM_EMB), jnp.float32),
        pltpu.SemaphoreType.DMA,
        pltpu.SemaphoreType.DMA,
    ],
)
def _sc_gather(users_hbm, items_hbm, utab_hbm, itab_hbm, out_u, out_i,
               uidx, iidx, urows, irows, usem, isem):
    wid = lax.axis_index("s") * NC + lax.axis_index("c")
    pltpu.sync_copy(users_hbm.at[wid], uidx)
    pltpu.sync_copy(items_hbm.at[wid], iidx)
    copies = []
    for c in range(NCH):
        copies.append(pltpu.async_copy(
            utab_hbm.at[uidx.at[c]], urows.at[pl.ds(c * CH, CH)], usem))
        copies.append(pltpu.async_copy(
            itab_hbm.at[iidx.at[c]], irows.at[pl.ds(c * CH, CH)], isem))
    for cp in copies:
        cp.wait()
    base = wid * BPW
    pltpu.sync_copy(urows, out_u.at[pl.ds(base, BPW)])
    pltpu.sync_copy(irows, out_i.at[pl.ds(base, BPW)])


BT = 2048


def _mlp_body(xu_ref, xi_ref,
              wl1u, wl1i, bl1, wl2, bl2, wl3, bl3,
              wr1u, wr1i, br1, wr2, br2, wr3, br3,
              ol_ref, or_ref):
    xu = xu_ref[...]
    xi = xi_ref[...]

    def head(w1u, w1i, b1, w2, b2, w3, b3):
        h = jnp.dot(xu, w1u[...], preferred_element_type=jnp.float32)
        h = h + jnp.dot(xi, w1i[...], preferred_element_type=jnp.float32)
        h = jnp.maximum(h + b1[...][None, :], 0.0)
        h = jnp.dot(h, w2[...], preferred_element_type=jnp.float32)
        h = jnp.maximum(h + b2[...][None, :], 0.0)
        o = jnp.sum(h * w3[...][None, :], axis=1) + b3[0]
        return 1.0 / (1.0 + jnp.exp(-o))

    ol_ref[...] = head(wl1u, wl1i, bl1, wl2, bl2, wl3, bl3)
    or_ref[...] = head(wr1u, wr1i, br1, wr2, br2, wr3, br3)


def _full(shape):
    return pl.BlockSpec(shape, lambda i: (0,) * len(shape))


_mlp_call = pl.pallas_call(
    _mlp_body,
    grid=(B // BT,),
    in_specs=[
        pl.BlockSpec((BT, EMB), lambda i: (i, 0)),
        pl.BlockSpec((BT, ITEM_EMB), lambda i: (i, 0)),
        _full((EMB, 32)), _full((ITEM_EMB, 32)), _full((32,)),
        _full((32, 16)), _full((16,)), _full((16,)), _full((1,)),
        _full((EMB, 32)), _full((ITEM_EMB, 32)), _full((32,)),
        _full((32, 16)), _full((16,)), _full((16,)), _full((1,)),
    ],
    out_specs=[
        pl.BlockSpec((BT,), lambda i: (i,)),
        pl.BlockSpec((BT,), lambda i: (i,)),
    ],
    out_shape=[
        jax.ShapeDtypeStruct((B,), jnp.float32),
        jax.ShapeDtypeStruct((B,), jnp.float32),
    ],
)


def kernel(users, items, user_embedding, item_embedding,
           Wl1, bl1, Wl2, bl2, Wl3, bl3,
           Wr1, br1, Wr2, br2, Wr3, br3):
    xu, xi = _sc_gather(users.reshape(NW, NCH, CH), items.reshape(NW, NCH, CH),
                        user_embedding, item_embedding)
    likes, rec = _mlp_call(
        xu, xi,
        Wl1[:EMB], Wl1[EMB:], bl1, Wl2, bl2, Wl3[:, 0], bl3,
        Wr1[:EMB], Wr1[EMB:], br1, Wr2, br2, Wr3[:, 0], br3)
    return likes, rec


# speedup vs baseline: 1.0459x; 1.0459x over previous
import functools

import jax
import jax.numpy as jnp
from jax import lax
from jax.experimental import pallas as pl
from jax.experimental.pallas import tpu as pltpu
from jax.experimental.pallas import tpu_sc as plsc

B = 16384
EMB = 64
ITEM_EMB = 24
NC, NS = 2, 16
NW = NC * NS
BPW = B // NW
CH = 128
NCH = BPW // CH

_sc_mesh = plsc.VectorSubcoreMesh(core_axis_name="c", subcore_axis_name="s")


@functools.partial(
    pl.kernel,
    out_type=(
        jax.ShapeDtypeStruct((B, EMB), jnp.bfloat16),
        jax.ShapeDtypeStruct((B, ITEM_EMB), jnp.float32),
    ),
    mesh=_sc_mesh,
    compiler_params=pltpu.CompilerParams(use_tc_tiling_on_sc=False),
    scratch_types=[
        pltpu.VMEM((NCH, CH), jnp.int32),
        pltpu.VMEM((NCH, CH), jnp.int32),
        pltpu.VMEM((BPW, EMB), jnp.bfloat16),
        pltpu.VMEM((BPW, ITEM_EMB), jnp.float32),
        pltpu.SemaphoreType.DMA,
        pltpu.SemaphoreType.DMA,
    ],
)
def _sc_gather(users_hbm, items_hbm, utab_hbm, itab_hbm, out_u, out_i,
               uidx, iidx, urows, irows, usem, isem):
    wid = lax.axis_index("s") * NC + lax.axis_index("c")
    pltpu.sync_copy(users_hbm.at[wid], uidx)
    pltpu.sync_copy(items_hbm.at[wid], iidx)
    copies = []
    for c in range(NCH):
        copies.append(pltpu.async_copy(
            utab_hbm.at[uidx.at[c]], urows.at[pl.ds(c * CH, CH)], usem))
        copies.append(pltpu.async_copy(
            itab_hbm.at[iidx.at[c]], irows.at[pl.ds(c * CH, CH)], isem))
    for cp in copies:
        cp.wait()
    base = wid * BPW
    pltpu.sync_copy(urows, out_u.at[pl.ds(base, BPW)])
    pltpu.sync_copy(irows, out_i.at[pl.ds(base, BPW)])


BT = 2048


def _mlp_body(xu_ref, xi_ref,
              wl1u, wl1i, bl1, wl2, bl2, wl3, bl3,
              wr1u, wr1i, br1, wr2, br2, wr3, br3,
              ol_ref, or_ref):
    xu = xu_ref[...].astype(jnp.float32)
    xi = xi_ref[...].astype(jnp.float32)

    def head(w1u, w1i, b1, w2, b2, w3, b3):
        h = jnp.dot(xu, w1u[...], preferred_element_type=jnp.float32)
        h = h + jnp.dot(xi, w1i[...], preferred_element_type=jnp.float32)
        h = jnp.maximum(h + b1[...][None, :], 0.0)
        h = jnp.dot(h, w2[...], preferred_element_type=jnp.float32)
        h = jnp.maximum(h + b2[...][None, :], 0.0)
        o = jnp.sum(h * w3[...][None, :], axis=1) + b3[0]
        return 1.0 / (1.0 + jnp.exp(-o))

    ol_ref[...] = head(wl1u, wl1i, bl1, wl2, bl2, wl3, bl3)
    or_ref[...] = head(wr1u, wr1i, br1, wr2, br2, wr3, br3)


def _full(shape):
    return pl.BlockSpec(shape, lambda i: (0,) * len(shape))


_mlp_call = pl.pallas_call(
    _mlp_body,
    grid=(B // BT,),
    in_specs=[
        pl.BlockSpec((BT, EMB), lambda i: (i, 0)),
        pl.BlockSpec((BT, ITEM_EMB), lambda i: (i, 0)),
        _full((EMB, 32)), _full((ITEM_EMB, 32)), _full((32,)),
        _full((32, 16)), _full((16,)), _full((16,)), _full((1,)),
        _full((EMB, 32)), _full((ITEM_EMB, 32)), _full((32,)),
        _full((32, 16)), _full((16,)), _full((16,)), _full((1,)),
    ],
    out_specs=[
        pl.BlockSpec((BT,), lambda i: (i,)),
        pl.BlockSpec((BT,), lambda i: (i,)),
    ],
    out_shape=[
        jax.ShapeDtypeStruct((B,), jnp.float32),
        jax.ShapeDtypeStruct((B,), jnp.float32),
    ],
)


def kernel(users, items, user_embedding, item_embedding,
           Wl1, bl1, Wl2, bl2, Wl3, bl3,
           Wr1, br1, Wr2, br2, Wr3, br3):
    utab = user_embedding.astype(jnp.bfloat16)
    itab = item_embedding
    xu, xi = _sc_gather(users.reshape(NW, NCH, CH), items.reshape(NW, NCH, CH),
                        utab, itab)
    likes, rec = _mlp_call(
        xu, xi,
        Wl1[:EMB], Wl1[EMB:], bl1, Wl2, bl2, Wl3[:, 0], bl3,
        Wr1[:EMB], Wr1[EMB:], br1, Wr2, br2, Wr3[:, 0], br3)
    return likes, rec


# speedup vs baseline: 3.6495x; 3.4895x over previous
import functools

import jax
import jax.numpy as jnp
from jax import lax
from jax.experimental import pallas as pl
from jax.experimental.pallas import tpu as pltpu
from jax.experimental.pallas import tpu_sc as plsc

B = 16384
EMB = 64
ITEM_EMB = 24
NC, NS = 2, 16
NW = NC * NS
BPW = B // NW
NB = B // 128
BLK_PER_W = NB // NW
TAIL_J = 1000000 // 128
DEPTH = 4

_sc_mesh = plsc.VectorSubcoreMesh(core_axis_name="c", subcore_axis_name="s")


@functools.partial(
    pl.kernel,
    out_type=(
        jax.ShapeDtypeStruct((NB, EMB, 128), jnp.float32),
        jax.ShapeDtypeStruct((NB, ITEM_EMB, 128), jnp.float32),
    ),
    mesh=_sc_mesh,
    compiler_params=pltpu.CompilerParams(needs_layout_passes=False),
    scratch_types=[
        pltpu.VMEM((BPW + 16,), jnp.int32),
        pltpu.VMEM((BPW + 16,), jnp.int32),
        pltpu.VMEM((DEPTH, EMB, 128), jnp.float32),
        pltpu.VMEM((DEPTH, ITEM_EMB, 128), jnp.float32),
        pltpu.VMEM((BLK_PER_W, EMB, 128), jnp.float32),
        pltpu.VMEM((BLK_PER_W, ITEM_EMB, 128), jnp.float32),
    ] + [pltpu.SemaphoreType.DMA] * (2 * DEPTH),
)
def _sc_gather(users_hbm, items_hbm, utab_t, itab_t, out_u, out_i,
               uidx_v, iidx_v, ubuf, ibuf, ustage, istage, *sems):
    usems, isems = sems[:DEPTH], sems[DEPTH:]
    wid = lax.axis_index("s") * NC + lax.axis_index("c")
    pltpu.sync_copy(users_hbm.at[wid], uidx_v.at[pl.ds(0, BPW)])
    pltpu.sync_copy(items_hbm.at[wid], iidx_v.at[pl.ds(0, BPW)])

    rows16 = jax.lax.iota(jnp.int32, 16)

    def start(vec, lane, tab, buf, sem, slot):
        off = pl.multiple_of((vec[lane] >> 7) * 128, 128)
        pltpu.async_copy(tab.at[:, pl.ds(off, 128)], buf.at[slot], sem)

    def wait(tab, buf, sem, slot):
        pltpu.make_async_copy(tab.at[:, pl.ds(0, 128)],
                              buf.at[slot], sem).wait()

    def extract(vec, lane, buf, stage, row_starts, i, slot):
        col = jnp.full((16,), vec[lane] & 127, dtype=jnp.int32)
        dst = jnp.full((16,), i & 127, dtype=jnp.int32)
        b = (i >> 7) & (BLK_PER_W - 1)
        for r0 in row_starts:
            rows = rows16 + r0
            v = plsc.load_gather(buf.at[slot], [rows, col])
            plsc.store_scatter(stage.at[b], [rows, dst], v)

    u0 = uidx_v[pl.ds(0, 16)]
    i0v = iidx_v[pl.ds(0, 16)]
    for d in range(DEPTH):
        start(u0, d, utab_t, ubuf, usems[d], d)
        start(i0v, d, itab_t, ibuf, isems[d], d)

    def outer(g, carry):
        base_i = g * 16
        ucur = uidx_v[pl.ds(base_i, 16)]
        unext = uidx_v[pl.ds(base_i + 16, 16)]
        icur = iidx_v[pl.ds(base_i, 16)]
        inext = iidx_v[pl.ds(base_i + 16, 16)]
        for l in range(16):
            d = l % DEPTH
            i = base_i + l
            wait(utab_t, ubuf, usems[d], d)
            extract(ucur, l, ubuf, ustage, (0, 16, 32, 48), i, d)
            wait(itab_t, ibuf, isems[d], d)
            extract(icur, l, ibuf, istage, (0, 8), i, d)
            ln = (l + DEPTH) % 16
            uv = ucur if l + DEPTH < 16 else unext
            iv = icur if l + DEPTH < 16 else inext

            @pl.when(i + DEPTH < BPW)
            def _(uv=uv, iv=iv, ln=ln, d=d):
                start(uv, ln, utab_t, ubuf, usems[d], d)
                start(iv, ln, itab_t, ibuf, isems[d], d)
        return carry

    lax.fori_loop(0, BPW // 16, outer, 0)

    for b in range(BLK_PER_W):
        pltpu.sync_copy(ustage.at[b], out_u.at[wid * BLK_PER_W + b])
        pltpu.sync_copy(istage.at[b], out_i.at[wid * BLK_PER_W + b])


BT = 2048


def _mlp_body(xu_ref, xi_ref,
              wl1u, wl1i, bl1, wl2, bl2, wl3, bl3,
              wr1u, wr1i, br1, wr2, br2, wr3, br3,
              ol_ref, or_ref):
    xu = xu_ref[...]
    xi = xi_ref[...]

    def head(w1u, w1i, b1, w2, b2, w3, b3):
        h = jnp.dot(xu, w1u[...], preferred_element_type=jnp.float32)
        h = h + jnp.dot(xi, w1i[...], preferred_element_type=jnp.float32)
        h = jnp.maximum(h + b1[...][None, :], 0.0)
        h = jnp.dot(h, w2[...], preferred_element_type=jnp.float32)
        h = jnp.maximum(h + b2[...][None, :], 0.0)
        o = jnp.sum(h * w3[...][None, :], axis=1) + b3[0]
        return 1.0 / (1.0 + jnp.exp(-o))

    ol_ref[...] = head(wl1u, wl1i, bl1, wl2, bl2, wl3, bl3)
    or_ref[...] = head(wr1u, wr1i, br1, wr2, br2, wr3, br3)


def _full(shape):
    return pl.BlockSpec(shape, lambda i: (0,) * len(shape))


_mlp_call = pl.pallas_call(
    _mlp_body,
    grid=(B // BT,),
    in_specs=[
        pl.BlockSpec((BT, EMB), lambda i: (i, 0)),
        pl.BlockSpec((BT, ITEM_EMB), lambda i: (i, 0)),
        _full((EMB, 32)), _full((ITEM_EMB, 32)), _full((32,)),
        _full((32, 16)), _full((16,)), _full((16,)), _full((1,)),
        _full((EMB, 32)), _full((ITEM_EMB, 32)), _full((32,)),
        _full((32, 16)), _full((16,)), _full((16,)), _full((1,)),
    ],
    out_specs=[
        pl.BlockSpec((BT,), lambda i: (i,)),
        pl.BlockSpec((BT,), lambda i: (i,)),
    ],
    out_shape=[
        jax.ShapeDtypeStruct((B,), jnp.float32),
        jax.ShapeDtypeStruct((B,), jnp.float32),
    ],
)


def kernel(users, items, user_embedding, item_embedding,
           Wl1, bl1, Wl2, bl2, Wl3, bl3,
           Wr1, br1, Wr2, br2, Wr3, br3):
    xu3, xi3 = _sc_gather(users.reshape(NW, BPW), items.reshape(NW, BPW),
                          user_embedding.T, item_embedding.T)
    xu = jnp.transpose(xu3, (0, 2, 1)).reshape(B, EMB)
    xi = jnp.transpose(xi3, (0, 2, 1)).reshape(B, ITEM_EMB)
    likes, rec = _mlp_call(
        xu, xi,
        Wl1[:EMB], Wl1[EMB:], bl1, Wl2, bl2, Wl3[:, 0], bl3,
        Wr1[:EMB], Wr1[EMB:], br1, Wr2, br2, Wr3[:, 0], br3)
    return likes, rec


# speedup vs baseline: 4.1322x; 1.1323x over previous
import functools

import jax
import jax.numpy as jnp
from jax import lax
from jax.experimental import pallas as pl
from jax.experimental.pallas import tpu as pltpu
from jax.experimental.pallas import tpu_sc as plsc

B = 16384
EMB = 64
ITEM_EMB = 24
NC, NS = 2, 16
NW = NC * NS
BPW = B // NW
NB = B // 128
BLK_PER_W = NB // NW
TAIL_J = 1000000 // 128
DEPTH = 4

_sc_mesh = plsc.VectorSubcoreMesh(core_axis_name="c", subcore_axis_name="s")


@functools.partial(
    pl.kernel,
    out_type=(
        jax.ShapeDtypeStruct((NB, EMB, 128), jnp.float32),
        jax.ShapeDtypeStruct((NB, ITEM_EMB, 128), jnp.float32),
    ),
    mesh=_sc_mesh,
    compiler_params=pltpu.CompilerParams(needs_layout_passes=False),
    scratch_types=[
        pltpu.VMEM((BPW + 16,), jnp.int32),
        pltpu.VMEM((BPW + 16,), jnp.int32),
        pltpu.VMEM((DEPTH, EMB, 128), jnp.float32),
        pltpu.VMEM((DEPTH, ITEM_EMB, 128), jnp.float32),
        pltpu.VMEM((BLK_PER_W, EMB, 128), jnp.float32),
        pltpu.VMEM((BLK_PER_W, ITEM_EMB, 128), jnp.float32),
    ] + [pltpu.SemaphoreType.DMA] * (2 * DEPTH),
)
def _sc_gather(users_hbm, items_hbm, utab_t, itab_t, out_u, out_i,
               uidx_v, iidx_v, ubuf, ibuf, ustage, istage, *sems):
    usems, isems = sems[:DEPTH], sems[DEPTH:]
    wid = lax.axis_index("s") * NC + lax.axis_index("c")
    pltpu.sync_copy(users_hbm.at[wid], uidx_v.at[pl.ds(0, BPW)])
    pltpu.sync_copy(items_hbm.at[wid], iidx_v.at[pl.ds(0, BPW)])

    rows16 = jax.lax.iota(jnp.int32, 16)

    def start(vec, lane, tab, buf, sem, slot):
        off = pl.multiple_of((vec[lane] >> 7) * 128, 128)
        pltpu.async_copy(tab.at[:, pl.ds(off, 128)], buf.at[slot], sem)

    def wait(tab, buf, sem, slot):
        pltpu.make_async_copy(tab.at[:, pl.ds(0, 128)],
                              buf.at[slot], sem).wait()

    def extract(vec, lane, buf, stage, row_starts, i, slot):
        col = jnp.full((16,), vec[lane] & 127, dtype=jnp.int32)
        dst = jnp.full((16,), i & 127, dtype=jnp.int32)
        b = (i >> 7) & (BLK_PER_W - 1)
        for r0 in row_starts:
            rows = rows16 + r0
            v = plsc.load_gather(buf.at[slot], [rows, col])
            plsc.store_scatter(stage.at[b], [rows, dst], v)

    u0 = uidx_v[pl.ds(0, 16)]
    i0v = iidx_v[pl.ds(0, 16)]
    for d in range(DEPTH):
        start(u0, d, utab_t, ubuf, usems[d], d)
        start(i0v, d, itab_t, ibuf, isems[d], d)

    def outer(g, carry):
        base_i = g * 16
        ucur = uidx_v[pl.ds(base_i, 16)]
        unext = uidx_v[pl.ds(base_i + 16, 16)]
        icur = iidx_v[pl.ds(base_i, 16)]
        inext = iidx_v[pl.ds(base_i + 16, 16)]
        for l in range(16):
            d = l % DEPTH
            i = base_i + l
            wait(utab_t, ubuf, usems[d], d)
            extract(ucur, l, ubuf, ustage, (0, 16, 32, 48), i, d)
            wait(itab_t, ibuf, isems[d], d)
            extract(icur, l, ibuf, istage, (0, 8), i, d)
            ln = (l + DEPTH) % 16
            uv = ucur if l + DEPTH < 16 else unext
            iv = icur if l + DEPTH < 16 else inext

            @pl.when(i + DEPTH < BPW)
            def _(uv=uv, iv=iv, ln=ln, d=d):
                start(uv, ln, utab_t, ubuf, usems[d], d)
                start(iv, ln, itab_t, ibuf, isems[d], d)
        return carry

    lax.fori_loop(0, BPW // 16, outer, 0)

    for b in range(BLK_PER_W):
        pltpu.sync_copy(ustage.at[b], out_u.at[wid * BLK_PER_W + b])
        pltpu.sync_copy(istage.at[b], out_i.at[wid * BLK_PER_W + b])


BT = 2048


BLK_PER_STEP = 16


def _mlp_body(xu_ref, xi_ref,
              wl1u, wl1i, bl1, wl2, bl2, wl3, bl3,
              wr1u, wr1i, br1, wr2, br2, wr3, br3,
              ol_ref, or_ref):
    xu = xu_ref[...]
    xi = xi_ref[...]

    def head(w1u, w1i, b1, w2, b2, w3, b3):
        h = lax.dot_general(xu, w1u[...], (((1,), (0,)), ((), ())),
                            preferred_element_type=jnp.float32)
        h = h + lax.dot_general(xi, w1i[...], (((1,), (0,)), ((), ())),
                                preferred_element_type=jnp.float32)
        h = jnp.maximum(h + b1[...][None, None, :], 0.0)
        h = lax.dot_general(h, w2[...], (((2,), (0,)), ((), ())),
                            preferred_element_type=jnp.float32)
        h = jnp.maximum(h + b2[...][None, None, :], 0.0)
        o = jnp.sum(h * w3[...][None, None, :], axis=2) + b3[0]
        return 1.0 / (1.0 + jnp.exp(-o))

    ol_ref[...] = head(wl1u, wl1i, bl1, wl2, bl2, wl3, bl3)
    or_ref[...] = head(wr1u, wr1i, br1, wr2, br2, wr3, br3)


def _full(shape):
    return pl.BlockSpec(shape, lambda i: (0,) * len(shape))


_mlp_call = pl.pallas_call(
    _mlp_body,
    grid=(NB // BLK_PER_STEP,),
    in_specs=[
        pl.BlockSpec((BLK_PER_STEP, EMB, 128), lambda i: (i, 0, 0)),
        pl.BlockSpec((BLK_PER_STEP, ITEM_EMB, 128), lambda i: (i, 0, 0)),
        _full((EMB, 32)), _full((ITEM_EMB, 32)), _full((32,)),
        _full((32, 16)), _full((16,)), _full((16,)), _full((1,)),
        _full((EMB, 32)), _full((ITEM_EMB, 32)), _full((32,)),
        _full((32, 16)), _full((16,)), _full((16,)), _full((1,)),
    ],
    out_specs=[
        pl.BlockSpec((BLK_PER_STEP, 128), lambda i: (i, 0)),
        pl.BlockSpec((BLK_PER_STEP, 128), lambda i: (i, 0)),
    ],
    out_shape=[
        jax.ShapeDtypeStruct((NB, 128), jnp.float32),
        jax.ShapeDtypeStruct((NB, 128), jnp.float32),
    ],
)


def kernel(users, items, user_embedding, item_embedding,
           Wl1, bl1, Wl2, bl2, Wl3, bl3,
           Wr1, br1, Wr2, br2, Wr3, br3):
    xu3, xi3 = _sc_gather(users.reshape(NW, BPW), items.reshape(NW, BPW),
                          user_embedding.T, item_embedding.T)
    likes2, rec2 = _mlp_call(
        xu3, xi3,
        Wl1[:EMB], Wl1[EMB:], bl1, Wl2, bl2, Wl3[:, 0], bl3,
        Wr1[:EMB], Wr1[EMB:], br1, Wr2, br2, Wr3[:, 0], br3)
    return likes2.reshape(B), rec2.reshape(B)


# speedup vs baseline: 4.1542x; 1.0053x over previous
import functools

import jax
import jax.numpy as jnp
from jax import lax
from jax.experimental import pallas as pl
from jax.experimental.pallas import tpu as pltpu
from jax.experimental.pallas import tpu_sc as plsc

B = 16384
EMB = 64
ITEM_EMB = 24
NC, NS = 2, 16
NW = NC * NS
BPW = B // NW
NB = B // 128
BLK_PER_W = NB // NW
TAIL_J = 1000000 // 128
DEPTH = 8

_sc_mesh = plsc.VectorSubcoreMesh(core_axis_name="c", subcore_axis_name="s")


@functools.partial(
    pl.kernel,
    out_type=(
        jax.ShapeDtypeStruct((NB, EMB, 128), jnp.float32),
        jax.ShapeDtypeStruct((NB, ITEM_EMB, 128), jnp.float32),
    ),
    mesh=_sc_mesh,
    compiler_params=pltpu.CompilerParams(needs_layout_passes=False),
    scratch_types=[
        pltpu.VMEM((BPW + 16,), jnp.int32),
        pltpu.VMEM((BPW + 16,), jnp.int32),
        pltpu.VMEM((DEPTH, EMB, 128), jnp.float32),
        pltpu.VMEM((DEPTH, ITEM_EMB, 128), jnp.float32),
        pltpu.VMEM((EMB, 128), jnp.float32),
        pltpu.VMEM((ITEM_EMB, 128), jnp.float32),
    ] + [pltpu.SemaphoreType.DMA] * (2 * DEPTH),
)
def _sc_gather(users_hbm, items_hbm, utab_t, itab_t, out_u, out_i,
               uidx_v, iidx_v, ubuf, ibuf, ustage, istage, *sems):
    usems, isems = sems[:DEPTH], sems[DEPTH:]
    wid = lax.axis_index("s") * NC + lax.axis_index("c")
    pltpu.sync_copy(users_hbm.at[wid], uidx_v.at[pl.ds(0, BPW)])
    pltpu.sync_copy(items_hbm.at[wid], iidx_v.at[pl.ds(0, BPW)])

    rows16 = jax.lax.iota(jnp.int32, 16)

    def start(vec, lane, tab, buf, sem, slot):
        off = pl.multiple_of((vec[lane] >> 7) * 128, 128)
        pltpu.async_copy(tab.at[:, pl.ds(off, 128)], buf.at[slot], sem)

    def wait(tab, buf, sem, slot):
        pltpu.make_async_copy(tab.at[:, pl.ds(0, 128)],
                              buf.at[slot], sem).wait()

    def extract(vec, lane, buf, stage, row_starts, i, slot):
        col = jnp.full((16,), vec[lane] & 127, dtype=jnp.int32)
        dst = jnp.full((16,), i & 127, dtype=jnp.int32)
        for r0 in row_starts:
            rows = rows16 + r0
            v = plsc.load_gather(buf.at[slot], [rows, col])
            plsc.store_scatter(stage, [rows, dst], v)

    u0 = uidx_v[pl.ds(0, 16)]
    i0v = iidx_v[pl.ds(0, 16)]
    for d in range(DEPTH):
        start(u0, d, utab_t, ubuf, usems[d], d)
        start(i0v, d, itab_t, ibuf, isems[d], d)

    def grp(b):
        def body(g, carry):
            base_i = b * 128 + g * 16
            ucur = uidx_v[pl.ds(base_i, 16)]
            unext = uidx_v[pl.ds(base_i + 16, 16)]
            icur = iidx_v[pl.ds(base_i, 16)]
            inext = iidx_v[pl.ds(base_i + 16, 16)]
            for l in range(16):
                d = l % DEPTH
                i = base_i + l
                wait(utab_t, ubuf, usems[d], d)
                extract(ucur, l, ubuf, ustage, (0, 16, 32, 48), i, d)
                wait(itab_t, ibuf, isems[d], d)
                extract(icur, l, ibuf, istage, (0, 8), i, d)
                ln = (l + DEPTH) % 16
                uv = ucur if l + DEPTH < 16 else unext
                iv = icur if l + DEPTH < 16 else inext

                @pl.when(i + DEPTH < BPW)
                def _(uv=uv, iv=iv, ln=ln, d=d):
                    start(uv, ln, utab_t, ubuf, usems[d], d)
                    start(iv, ln, itab_t, ibuf, isems[d], d)
            return carry

        return body

    for b in range(BLK_PER_W):
        lax.fori_loop(0, 8, grp(b), 0)
        pltpu.sync_copy(ustage, out_u.at[wid * BLK_PER_W + b])
        pltpu.sync_copy(istage, out_i.at[wid * BLK_PER_W + b])


BT = 2048


BLK_PER_STEP = 16


def _mlp_body(xu_ref, xi_ref,
              wl1u, wl1i, bl1, wl2, bl2, wl3, bl3,
              wr1u, wr1i, br1, wr2, br2, wr3, br3,
              ol_ref, or_ref):
    xu = xu_ref[...]
    xi = xi_ref[...]

    def head(w1u, w1i, b1, w2, b2, w3, b3):
        h = lax.dot_general(xu, w1u[...], (((1,), (0,)), ((), ())),
                            preferred_element_type=jnp.float32)
        h = h + lax.dot_general(xi, w1i[...], (((1,), (0,)), ((), ())),
                                preferred_element_type=jnp.float32)
        h = jnp.maximum(h + b1[...][None, None, :], 0.0)
        h = lax.dot_general(h, w2[...], (((2,), (0,)), ((), ())),
                            preferred_element_type=jnp.float32)
        h = jnp.maximum(h + b2[...][None, None, :], 0.0)
        o = jnp.sum(h * w3[...][None, None, :], axis=2) + b3[0]
        return 1.0 / (1.0 + jnp.exp(-o))

    ol_ref[...] = head(wl1u, wl1i, bl1, wl2, bl2, wl3, bl3)
    or_ref[...] = head(wr1u, wr1i, br1, wr2, br2, wr3, br3)


def _full(shape):
    return pl.BlockSpec(shape, lambda i: (0,) * len(shape))


_mlp_call = pl.pallas_call(
    _mlp_body,
    grid=(NB // BLK_PER_STEP,),
    in_specs=[
        pl.BlockSpec((BLK_PER_STEP, EMB, 128), lambda i: (i, 0, 0)),
        pl.BlockSpec((BLK_PER_STEP, ITEM_EMB, 128), lambda i: (i, 0, 0)),
        _full((EMB, 32)), _full((ITEM_EMB, 32)), _full((32,)),
        _full((32, 16)), _full((16,)), _full((16,)), _full((1,)),
        _full((EMB, 32)), _full((ITEM_EMB, 32)), _full((32,)),
        _full((32, 16)), _full((16,)), _full((16,)), _full((1,)),
    ],
    out_specs=[
        pl.BlockSpec((BLK_PER_STEP, 128), lambda i: (i, 0)),
        pl.BlockSpec((BLK_PER_STEP, 128), lambda i: (i, 0)),
    ],
    out_shape=[
        jax.ShapeDtypeStruct((NB, 128), jnp.float32),
        jax.ShapeDtypeStruct((NB, 128), jnp.float32),
    ],
)


def kernel(users, items, user_embedding, item_embedding,
           Wl1, bl1, Wl2, bl2, Wl3, bl3,
           Wr1, br1, Wr2, br2, Wr3, br3):
    xu3, xi3 = _sc_gather(users.reshape(NW, BPW), items.reshape(NW, BPW),
                          user_embedding.T, item_embedding.T)
    likes2, rec2 = _mlp_call(
        xu3, xi3,
        Wl1[:EMB], Wl1[EMB:], bl1, Wl2, bl2, Wl3[:, 0], bl3,
        Wr1[:EMB], Wr1[EMB:], br1, Wr2, br2, Wr3[:, 0], br3)
    return likes2.reshape(B), rec2.reshape(B)
